# baseline (speedup 1.0000x reference)
import functools

import jax
import jax.numpy as jnp
from jax import lax
from jax.experimental import pallas as pl
from jax.experimental.pallas import tpu as pltpu
from jax.experimental.pallas import tpu_sc as plsc

BATCH = 16384
RANK = 32
SIZE = 100000
NC, NS, L = 2, 16, 16
NW = NC * NS
BPW = BATCH // NW
HALF = BPW // 2
NFULL = SIZE // 128
TAIL = NFULL * 128
BLK_F32 = 128 * RANK


def _conv_body(t0, t1, t2, tcat, g0, g1, g2,
               tin_v, stage_v, sin, sout):
    wid = lax.axis_index("s") * NC + lax.axis_index("c")
    lo = wid * NFULL // NW
    hi = (wid + 1) * NFULL // NW
    n = hi - lo

    @pl.when(wid == 0)
    def _():
        for k, dst in enumerate((g0, g1, g2)):
            pltpu.sync_copy(tcat.at[pl.ds(k * 1024, 1024)],
                            dst.at[pl.ds(TAIL * RANK, 1024)])

    lanes = lax.iota(jnp.int32, L)
    bi = [lanes * RANK + lg * L * RANK for lg in range(8)]

    NB = 6

    def in_copy(src, bid, b):
        return pltpu.make_async_copy(
            src.at[:, :, pl.ds(bid * 128, 128)],
            tin_v.at[pl.ds(b * 4, 4)], sin.at[b])

    def out_copy(dst, bid, b):
        return pltpu.make_async_copy(
            stage_v.at[pl.ds(b * BLK_F32, BLK_F32)],
            dst.at[pl.ds(bid * BLK_F32, BLK_F32)], sout.at[b])

    for src, dst in ((t0, g0), (t1, g1), (t2, g2)):
        for p in range(NB):
            in_copy(src, lo + p, p).start()

        def blk(g, carry):
            bid = lo + g
            b = g % NB
            bs = g % 2

            @pl.when(g >= 2)
            def _():
                out_copy(dst, bid - 2, bs).wait()

            in_copy(src, bid, b).wait()
            stage = stage_v.at[pl.ds(bs * BLK_F32, BLK_F32)]
            for a in range(4):
                for s in range(8):
                    base = a * 8 + s
                    vs = [tin_v[b * 4 + a, s, pl.ds(lg * L, L)]
                          for lg in range(8)]
                    for lg in range(8):
                        plsc.store_scatter(stage, [bi[lg] + base], vs[lg])
            out_copy(dst, bid, bs).start()

            @pl.when(g + NB < n)
            def _():
                in_copy(src, bid + NB, b).start()
            return carry

        lax.fori_loop(0, n, blk, 0)
        out_copy(dst, lo, 0).wait()
        out_copy(dst, lo, 1).wait()


_convert = functools.partial(
    pl.kernel,
    out_type=(jax.ShapeDtypeStruct((SIZE * RANK,), jnp.float32),) * 3,
    mesh=plsc.VectorSubcoreMesh(core_axis_name="c", subcore_axis_name="s",
                                num_cores=NC, num_subcores=NS),
    compiler_params=pltpu.CompilerParams(needs_layout_passes=False,
                                         use_tc_tiling_on_sc=True),
    scratch_types=[
        pltpu.VMEM((24, 8, 128), jnp.float32),
        pltpu.VMEM((2 * BLK_F32,), jnp.float32),
        pltpu.SemaphoreType.DMA((6,)),
        pltpu.SemaphoreType.DMA((2,)),
    ],
)(_conv_body)


def _cp_body(idx_hbm, lamb_hbm, f0_hbm, f1_hbm, f2_hbm, out_hbm,
             i_v, g0_v, g1_v, g2_v, lamb_v, out_v, sem0, sem1):
    wid = lax.axis_index("s") * NC + lax.axis_index("c")
    base = wid * BPW

    pltpu.sync_copy(idx_hbm.at[:, pl.ds(base, BPW)], i_v)

    copies = []
    for half, sem in ((0, sem0), (1, sem1)):
        sl = pl.ds(half * HALF, HALF)
        for k, (f_hbm, g_v) in enumerate(
                ((f0_hbm, g0_v), (f1_hbm, g1_v), (f2_hbm, g2_v))):
            c = pltpu.make_async_copy(f_hbm.at[i_v.at[k, sl]], g_v.at[sl], sem)
            c.start()
            copies.append(c)

    pltpu.sync_copy(lamb_hbm, lamb_v)
    lam = [plsc.load_gather(lamb_v, [jnp.full((L,), j, jnp.int32)])
           for j in range(RANK)]
    lanes = lax.iota(jnp.int32, L)

    def group(g, carry):
        rows = lanes + g * L
        accs = [jnp.zeros((L,), jnp.float32) for _ in range(4)]
        for j in range(RANK):
            col = jnp.full((L,), j, jnp.int32)
            v0 = plsc.load_gather(g0_v, [rows, col])
            v1 = plsc.load_gather(g1_v, [rows, col])
            v2 = plsc.load_gather(g2_v, [rows, col])
            accs[j % 4] = accs[j % 4] + (v0 * v1) * (v2 * lam[j])
        out_v[pl.ds(g * L, L)] = (accs[0] + accs[1]) + (accs[2] + accs[3])
        return carry

    copies[0].wait()
    copies[1].wait()
    copies[2].wait()
    lax.fori_loop(0, HALF // L, group, 0)
    copies[3].wait()
    copies[4].wait()
    copies[5].wait()
    lax.fori_loop(HALF // L, BPW // L, group, 0)

    pltpu.sync_copy(out_v, out_hbm.at[pl.ds(base, BPW)])


_cp_kernel = functools.partial(
    pl.kernel,
    out_type=jax.ShapeDtypeStruct((BATCH,), jnp.float32),
    mesh=plsc.VectorSubcoreMesh(core_axis_name="c", subcore_axis_name="s",
                                num_cores=NC, num_subcores=NS),
    compiler_params=pltpu.CompilerParams(needs_layout_passes=False,
                                         use_tc_tiling_on_sc=False),
    scratch_types=[
        pltpu.VMEM((3, BPW), jnp.int32),
        pltpu.VMEM((BPW, RANK), jnp.float32),
        pltpu.VMEM((BPW, RANK), jnp.float32),
        pltpu.VMEM((BPW, RANK), jnp.float32),
        pltpu.VMEM((RANK,), jnp.float32),
        pltpu.VMEM((BPW,), jnp.float32),
        pltpu.SemaphoreType.DMA,
        pltpu.SemaphoreType.DMA,
    ],
)(_cp_body)


def kernel(input, lamb, f0, f1, f2):
    idx = input.astype(jnp.int32)
    idx_t = idx.T

    t0, t1, t2 = (f.T.reshape(4, 8, SIZE) for f in (f0, f1, f2))
    tcat = jnp.concatenate(
        [f[TAIL:, :].reshape(-1) for f in (f0, f1, f2)])
    g0, g1, g2 = _convert(t0, t1, t2, tcat)

    return _cp_kernel(idx_t, lamb,
                      g0.reshape(SIZE, RANK),
                      g1.reshape(SIZE, RANK),
                      g2.reshape(SIZE, RANK))


# speedup vs baseline: 1.3832x; 1.3832x over previous
import functools

import jax
import jax.numpy as jnp
from jax import lax
from jax.experimental import pallas as pl
from jax.experimental.pallas import tpu as pltpu
from jax.experimental.pallas import tpu_sc as plsc

BATCH = 16384
RANK = 32
NC, NS, L = 2, 16, 16
NW = NC * NS
BPW = BATCH // NW
HALF = BPW // 2


def _cp_body(idx_hbm, lamb_hbm, f0_hbm, f1_hbm, f2_hbm, out_hbm,
             i_v, g0_v, g1_v, g2_v, lamb_v, out_v, sem0, sem1):
    wid = lax.axis_index("s") * NC + lax.axis_index("c")
    base = wid * BPW

    pltpu.sync_copy(idx_hbm.at[:, pl.ds(base, BPW)], i_v)

    copies = []
    for half, sem in ((0, sem0), (1, sem1)):
        sl = pl.ds(half * HALF, HALF)
        for k, (f_hbm, g_v) in enumerate(
                ((f0_hbm, g0_v), (f1_hbm, g1_v), (f2_hbm, g2_v))):
            c = pltpu.make_async_copy(f_hbm.at[i_v.at[k, sl]], g_v.at[sl], sem)
            c.start()
            copies.append(c)

    pltpu.sync_copy(lamb_hbm, lamb_v)
    lam = [plsc.load_gather(lamb_v, [jnp.full((L,), j, jnp.int32)])
           for j in range(RANK)]
    lanes = lax.iota(jnp.int32, L)

    def group(g, carry):
        rows = lanes + g * L
        accs = [jnp.zeros((L,), jnp.float32) for _ in range(4)]
        for j in range(RANK):
            col = jnp.full((L,), j, jnp.int32)
            v0 = plsc.load_gather(g0_v, [rows, col])
            v1 = plsc.load_gather(g1_v, [rows, col])
            v2 = plsc.load_gather(g2_v, [rows, col])
            accs[j % 4] = accs[j % 4] + (v0 * v1) * (v2 * lam[j])
        out_v[pl.ds(g * L, L)] = (accs[0] + accs[1]) + (accs[2] + accs[3])
        return carry

    copies[0].wait()
    copies[1].wait()
    copies[2].wait()
    lax.fori_loop(0, HALF // L, group, 0)
    copies[3].wait()
    copies[4].wait()
    copies[5].wait()
    lax.fori_loop(HALF // L, BPW // L, group, 0)

    pltpu.sync_copy(out_v, out_hbm.at[pl.ds(base, BPW)])


_cp_kernel = functools.partial(
    pl.kernel,
    out_type=jax.ShapeDtypeStruct((BATCH,), jnp.float32),
    mesh=plsc.VectorSubcoreMesh(core_axis_name="c", subcore_axis_name="s",
                                num_cores=NC, num_subcores=NS),
    compiler_params=pltpu.CompilerParams(needs_layout_passes=False,
                                         use_tc_tiling_on_sc=False),
    scratch_types=[
        pltpu.VMEM((3, BPW), jnp.int32),
        pltpu.VMEM((BPW, RANK), jnp.float32),
        pltpu.VMEM((BPW, RANK), jnp.float32),
        pltpu.VMEM((BPW, RANK), jnp.float32),
        pltpu.VMEM((RANK,), jnp.float32),
        pltpu.VMEM((BPW,), jnp.float32),
        pltpu.SemaphoreType.DMA,
        pltpu.SemaphoreType.DMA,
    ],
)(_cp_body)


def kernel(input, lamb, f0, f1, f2):
    idx_t = input.astype(jnp.int32).T
    return _cp_kernel(idx_t, lamb, f0, f1, f2)
